# baseline (speedup 1.0000x reference)
import functools

import jax
import jax.numpy as jnp
from jax import lax
from jax.experimental import pallas as pl
from jax.experimental.pallas import tpu as pltpu
from jax.experimental.pallas import tpu_sc as plsc

N = 10000
D = 128
E = 320000

NC = 2
NS = 16
CH = 128
DW = 64
CW = 16
NPAD = 10240
RPT = NPAD // NS
SEGC = 16


def _ceil_to(x, m):
    return (x + m - 1) // m * m


CPT = _ceil_to(_ceil_to(E // NS, CH) // CH, SEGC)
PC = CPT * CH * NS


def _make_sweep(dw, aux):
    mesh = plsc.VectorSubcoreMesh(core_axis_name="c", subcore_axis_name="s")
    out_type = [jax.ShapeDtypeStruct((NC, NPAD, dw), jnp.float32)]
    scratch = [
        pltpu.VMEM((SEGC, CH), jnp.int32),
        pltpu.VMEM((SEGC, CH), jnp.int32),
        pltpu.VMEM((CH, dw), jnp.float32),
        pltpu.VMEM((CH, dw), jnp.float32),
        pltpu.VMEM((16, dw), jnp.float32),
        pltpu.VMEM_SHARED((NPAD, dw), jnp.float32),
        pltpu.SemaphoreType.DMA,
        pltpu.SemaphoreType.DMA,
        pltpu.SemaphoreType.DMA,
        pltpu.SemaphoreType.DMA,
    ]
    if aux:
        out_type += [jax.ShapeDtypeStruct((NC, NPAD, CW), jnp.float32)]
        scratch += [
            pltpu.VMEM((CH, CW), jnp.float32),
            pltpu.VMEM((16, CW), jnp.float32),
            pltpu.VMEM_SHARED((NPAD, CW), jnp.float32),
            pltpu.SemaphoreType.DMA,
        ]

    @functools.partial(
        pl.kernel,
        out_type=out_type,
        mesh=mesh,
        scratch_types=scratch,
        compiler_params=pltpu.CompilerParams(use_tc_tiling_on_sc=False),
    )
    def sweep(gidx_hbm, sidx_hbm, table_hbm, *rest):
        if aux:
            (ones_hbm, out_hbm, oc_hbm, gidx_v, sidx_v, rowbuf0,
             rowbuf1, ztile, acc, gsem0, gsem1, ssem0, ssem1, onesbuf, zc,
             acc_c, csem) = rest
        else:
            (out_hbm, gidx_v, sidx_v, rowbuf0, rowbuf1, ztile,
             acc, gsem0, gsem1, ssem0, ssem1) = rest
        rowbufs = (rowbuf0, rowbuf1)
        gsems = (gsem0, gsem1)
        ssems = (ssem0, ssem1)
        c = lax.axis_index("c")
        s = lax.axis_index("s")
        base = s * RPT
        tbl_c = table_hbm.at[pl.ds(c * NPAD, NPAD)]

        zv = jnp.zeros((16,), jnp.float32)
        for r in range(16):
            for o in range(0, dw, 16):
                ztile[r, pl.ds(o, 16)] = zv
        zpend = [
            pltpu.async_copy(ztile, acc.at[pl.ds(base + i * 16, 16)], gsem0)
            for i in range(RPT // 16)
        ]
        if aux:
            for r in range(16):
                zc[r, pl.ds(0, CW)] = zv
            zpend += [pltpu.async_copy(ones_hbm, onesbuf, csem)]
            zpend += [
                pltpu.async_copy(zc, acc_c.at[pl.ds(base + i * 16, 16)],
                                 gsem1)
                for i in range(RPT // 16)
            ]
        for zp in zpend:
            zp.wait()
        plsc.subcore_barrier()

        def seg_body(si, carry):
            ip0 = pltpu.async_copy(
                gidx_hbm.at[s, pl.ds(si * SEGC, SEGC)], gidx_v, gsem0)
            ip1 = pltpu.async_copy(
                sidx_hbm.at[s, pl.ds(si * SEGC, SEGC)], sidx_v, gsem1)
            ip0.wait()
            ip1.wait()

            pend_g = [None, None]
            pend_s = [None, None]
            pend_g[0] = pltpu.async_copy(
                tbl_c.at[gidx_v.at[0]], rowbufs[0], gsems[0])
            for j in range(SEGC):
                cur = j & 1
                if j + 1 < SEGC:
                    if pend_s[1 - cur] is not None:
                        pend_s[1 - cur].wait()
                    pend_g[1 - cur] = pltpu.async_copy(
                        tbl_c.at[gidx_v.at[j + 1]],
                        rowbufs[1 - cur], gsems[1 - cur])
                pend_g[cur].wait()
                pend_s[cur] = pltpu.async_copy(
                    rowbufs[cur], acc.at[sidx_v.at[j]], ssems[cur],
                    add=True)
                if aux:
                    @pl.when(c == 0)
                    def _():
                        pltpu.async_copy(
                            onesbuf, acc_c.at[sidx_v.at[j]], csem, add=True)

                    @pl.when(c == 1)
                    def _():
                        pltpu.async_copy(
                            onesbuf, acc_c.at[gidx_v.at[j]], csem, add=True)
            pend_s[0].wait()
            pend_s[1].wait()
            if aux:
                for j in range(SEGC):
                    pltpu.make_async_copy(
                        onesbuf, acc_c.at[sidx_v.at[0]], csem).wait()
            return carry
        lax.fori_loop(0, CPT // SEGC, seg_body, 0)
        plsc.subcore_barrier()

        pltpu.sync_copy(acc.at[pl.ds(base, RPT)],
                        out_hbm.at[c, pl.ds(base, RPT)])
        if aux:
            pltpu.sync_copy(acc_c.at[pl.ds(base, RPT)],
                            oc_hbm.at[c, pl.ds(base, RPT)])

    return sweep


BN = 1024


def _dense_body(agg_ref, cnt_ref, x_ref, wn_ref, ws_ref, b_ref, wq_ref,
                bq_ref, a_ref, btab_ref):
    agg = jnp.concatenate([agg_ref[0], agg_ref[1]], axis=1)
    cnt = cnt_ref[0, :, 0:1]
    mean = agg / jnp.maximum(cnt, 1.0)
    h = jnp.maximum(
        jnp.dot(mean, wn_ref[...], preferred_element_type=jnp.float32)
        + jnp.dot(x_ref[...], ws_ref[...], preferred_element_type=jnp.float32)
        + b_ref[...], 0.0)
    a_ref[...] = jnp.dot(h, wq_ref[:D], preferred_element_type=jnp.float32)
    bmat = jnp.dot(h, wq_ref[D:], preferred_element_type=jnp.float32) + bq_ref[...]
    btab_ref[0] = jnp.concatenate(
        [bmat[:, :DW], bmat[:, :DW] * bmat[:, :DW]], axis=1)
    btab_ref[1] = jnp.concatenate(
        [bmat[:, DW:], bmat[:, DW:] * bmat[:, DW:]], axis=1)


def _final_body(a_ref, s_ref, cs_ref, out_ref):
    a = a_ref[...]
    cs = cs_ref[1, :, 0:1]
    sb = jnp.concatenate([s_ref[0, :, :DW], s_ref[1, :, :DW]], axis=1)
    sb2 = jnp.concatenate([s_ref[0, :, DW:], s_ref[1, :, DW:]], axis=1)
    gs = cs * a * a + 2.0 * a * sb + sb2
    out_ref[...] = jnp.tanh(gs / jnp.maximum(cs, 1.0))


def kernel(X, edge_index, Wn, Ws, b_sage, Wq, bq):
    src = edge_index[0]
    dst = edge_index[1]

    pad = jnp.full((PC - E,), N, jnp.int32)
    srcp = jnp.concatenate([src, pad]).reshape(NS, CPT, CH)
    dstp = jnp.concatenate([dst, pad]).reshape(NS, CPT, CH)

    t1 = jnp.zeros((NC, NPAD, DW), jnp.float32)
    t1 = t1.at[0, :N].set(X[:, :DW]).at[1, :N].set(X[:, DW:])
    t1 = t1.reshape(NC * NPAD, DW)

    ones_rows = jnp.zeros((CH, CW), jnp.float32).at[:, 0].set(1.0)

    sweep1 = _make_sweep(DW, aux=True)
    o1d, o1c = sweep1(srcp, dstp, t1, ones_rows)

    grid = (NPAD // BN,)
    a_full, btab = pl.pallas_call(
        _dense_body,
        grid=grid,
        in_specs=[
            pl.BlockSpec((NC, BN, DW), lambda i: (0, i, 0)),
            pl.BlockSpec((NC, BN, CW), lambda i: (0, i, 0)),
            pl.BlockSpec((BN, D), lambda i: (i, 0)),
            pl.BlockSpec((D, D), lambda i: (0, 0)),
            pl.BlockSpec((D, D), lambda i: (0, 0)),
            pl.BlockSpec((1, D), lambda i: (0, 0)),
            pl.BlockSpec((2 * D, D), lambda i: (0, 0)),
            pl.BlockSpec((1, D), lambda i: (0, 0)),
        ],
        out_specs=[
            pl.BlockSpec((BN, D), lambda i: (i, 0)),
            pl.BlockSpec((NC, BN, 2 * DW), lambda i: (0, i, 0)),
        ],
        out_shape=[
            jax.ShapeDtypeStruct((NPAD, D), jnp.float32),
            jax.ShapeDtypeStruct((NC, NPAD, 2 * DW), jnp.float32),
        ],
    )(o1d, o1c, jnp.zeros((NPAD, D), jnp.float32).at[:N].set(X),
      Wn, Ws, b_sage.reshape(1, D), Wq, bq.reshape(1, D))

    sweep2 = _make_sweep(2 * DW, aux=False)
    (o2,) = sweep2(dstp, srcp, btab.reshape(NC * NPAD, 2 * DW))

    gg = pl.pallas_call(
        _final_body,
        grid=grid,
        in_specs=[
            pl.BlockSpec((BN, D), lambda i: (i, 0)),
            pl.BlockSpec((NC, BN, 2 * DW), lambda i: (0, i, 0)),
            pl.BlockSpec((NC, BN, CW), lambda i: (0, i, 0)),
        ],
        out_specs=pl.BlockSpec((BN, D), lambda i: (i, 0)),
        out_shape=jax.ShapeDtypeStruct((NPAD, D), jnp.float32),
    )(a_full, o2, o1c)

    return gg[:N]


# speedup vs baseline: 1.2712x; 1.2712x over previous
import functools

import jax
import jax.numpy as jnp
from jax import lax
from jax.experimental import pallas as pl
from jax.experimental.pallas import tpu as pltpu
from jax.experimental.pallas import tpu_sc as plsc

N = 10000
D = 128
E = 320000

NC = 2
NS = 16
CH = 128
DW = 64
CW = 16
NPAD = 10240
RPT = NPAD // NS
SEGC = 16


def _ceil_to(x, m):
    return (x + m - 1) // m * m


CPT = _ceil_to(_ceil_to(E // NS, CH) // CH, SEGC)
PC = CPT * CH * NS


def _make_sweep(dw, aux, sq=False):
    mesh = plsc.VectorSubcoreMesh(core_axis_name="c", subcore_axis_name="s")
    out_type = [jax.ShapeDtypeStruct((NC, NPAD, dw), jnp.float32)] * (
        2 if sq else 1)
    scratch = [
        pltpu.VMEM((SEGC, CH), jnp.int32),
        pltpu.VMEM((SEGC, CH), jnp.int32),
        pltpu.VMEM((CH, dw), jnp.float32),
        pltpu.VMEM((CH, dw), jnp.float32),
        pltpu.VMEM((16, dw), jnp.float32),
        pltpu.VMEM_SHARED((NPAD, dw), jnp.float32),
        pltpu.SemaphoreType.DMA,
        pltpu.SemaphoreType.DMA,
        pltpu.SemaphoreType.DMA,
        pltpu.SemaphoreType.DMA,
    ]
    if sq:
        scratch += [
            pltpu.VMEM((CH, dw), jnp.float32),
            pltpu.VMEM((CH, dw), jnp.float32),
            pltpu.VMEM_SHARED((NPAD, dw), jnp.float32),
            pltpu.SemaphoreType.DMA,
            pltpu.SemaphoreType.DMA,
        ]
    if aux:
        out_type += [jax.ShapeDtypeStruct((NC, NPAD, CW), jnp.float32)]
        scratch += [
            pltpu.VMEM((CH, CW), jnp.float32),
            pltpu.VMEM((16, CW), jnp.float32),
            pltpu.VMEM_SHARED((NPAD, CW), jnp.float32),
            pltpu.SemaphoreType.DMA,
        ]

    @functools.partial(
        pl.kernel,
        out_type=out_type,
        mesh=mesh,
        scratch_types=scratch,
        compiler_params=pltpu.CompilerParams(use_tc_tiling_on_sc=False),
    )
    def sweep(gidx_hbm, sidx_hbm, table_hbm, *rest):
        if aux:
            (ones_hbm, out_hbm, oc_hbm, gidx_v, sidx_v, rowbuf0,
             rowbuf1, ztile, acc, gsem0, gsem1, ssem0, ssem1, onesbuf, zc,
             acc_c, csem) = rest
        elif sq:
            (out_hbm, oq_hbm, gidx_v, sidx_v, rowbuf0, rowbuf1, ztile,
             acc, gsem0, gsem1, ssem0, ssem1, sqbuf0, sqbuf1, acc_q,
             qsem0, qsem1) = rest
            sqbufs = (sqbuf0, sqbuf1)
            qsems = (qsem0, qsem1)
        else:
            (out_hbm, gidx_v, sidx_v, rowbuf0, rowbuf1, ztile,
             acc, gsem0, gsem1, ssem0, ssem1) = rest
        rowbufs = (rowbuf0, rowbuf1)
        gsems = (gsem0, gsem1)
        ssems = (ssem0, ssem1)
        c = lax.axis_index("c")
        s = lax.axis_index("s")
        base = s * RPT
        tbl_c = table_hbm.at[pl.ds(c * NPAD, NPAD)]

        zv = jnp.zeros((16,), jnp.float32)
        for r in range(16):
            for o in range(0, dw, 16):
                ztile[r, pl.ds(o, 16)] = zv
        zpend = [
            pltpu.async_copy(ztile, acc.at[pl.ds(base + i * 16, 16)], gsem0)
            for i in range(RPT // 16)
        ]
        if sq:
            zpend += [
                pltpu.async_copy(ztile, acc_q.at[pl.ds(base + i * 16, 16)],
                                 gsem1)
                for i in range(RPT // 16)
            ]
        if aux:
            for r in range(16):
                zc[r, pl.ds(0, CW)] = zv
            zpend += [pltpu.async_copy(ones_hbm, onesbuf, csem)]
            zpend += [
                pltpu.async_copy(zc, acc_c.at[pl.ds(base + i * 16, 16)],
                                 gsem1)
                for i in range(RPT // 16)
            ]
        for zp in zpend:
            zp.wait()
        plsc.subcore_barrier()

        def seg_body(si, carry):
            ip0 = pltpu.async_copy(
                gidx_hbm.at[s, pl.ds(si * SEGC, SEGC)], gidx_v, gsem0)
            ip1 = pltpu.async_copy(
                sidx_hbm.at[s, pl.ds(si * SEGC, SEGC)], sidx_v, gsem1)
            ip0.wait()
            ip1.wait()

            pend_g = [None, None]
            pend_s = [None, None]
            pend_q = [None, None]
            pend_g[0] = pltpu.async_copy(
                tbl_c.at[gidx_v.at[0]], rowbufs[0], gsems[0])
            for j in range(SEGC):
                cur = j & 1
                if j + 1 < SEGC:
                    if pend_s[1 - cur] is not None:
                        pend_s[1 - cur].wait()
                    if sq and pend_q[1 - cur] is not None:
                        pend_q[1 - cur].wait()
                    pend_g[1 - cur] = pltpu.async_copy(
                        tbl_c.at[gidx_v.at[j + 1]],
                        rowbufs[1 - cur], gsems[1 - cur])
                pend_g[cur].wait()
                if sq:
                    rb, qb = rowbufs[cur], sqbufs[cur]

                    def sq_body(r, carry):
                        for o in range(0, dw, 16):
                            v = rb[r, pl.ds(o, 16)]
                            qb[r, pl.ds(o, 16)] = v * v
                        return carry
                    lax.fori_loop(0, CH, sq_body, 0)
                    pend_q[cur] = pltpu.async_copy(
                        qb, acc_q.at[sidx_v.at[j]], qsems[cur], add=True)
                pend_s[cur] = pltpu.async_copy(
                    rowbufs[cur], acc.at[sidx_v.at[j]], ssems[cur],
                    add=True)
                if aux:
                    @pl.when(c == 0)
                    def _():
                        pltpu.async_copy(
                            onesbuf, acc_c.at[sidx_v.at[j]], csem, add=True)

                    @pl.when(c == 1)
                    def _():
                        pltpu.async_copy(
                            onesbuf, acc_c.at[gidx_v.at[j]], csem, add=True)
            pend_s[0].wait()
            pend_s[1].wait()
            if sq:
                if pend_q[0] is not None:
                    pend_q[0].wait()
                if pend_q[1] is not None:
                    pend_q[1].wait()
            if aux:
                for j in range(SEGC):
                    pltpu.make_async_copy(
                        onesbuf, acc_c.at[sidx_v.at[0]], csem).wait()
            return carry
        lax.fori_loop(0, CPT // SEGC, seg_body, 0)
        plsc.subcore_barrier()

        pltpu.sync_copy(acc.at[pl.ds(base, RPT)],
                        out_hbm.at[c, pl.ds(base, RPT)])
        if sq:
            pltpu.sync_copy(acc_q.at[pl.ds(base, RPT)],
                            oq_hbm.at[c, pl.ds(base, RPT)])
        if aux:
            pltpu.sync_copy(acc_c.at[pl.ds(base, RPT)],
                            oc_hbm.at[c, pl.ds(base, RPT)])

    return sweep


BN = 1024


def _dense_body(agg_ref, cnt_ref, x_ref, wn_ref, ws_ref, b_ref, wq_ref,
                bq_ref, a_ref, btab_ref):
    agg = jnp.concatenate([agg_ref[0], agg_ref[1]], axis=1)
    cnt = cnt_ref[0, :, 0:1]
    mean = agg / jnp.maximum(cnt, 1.0)
    h = jnp.maximum(
        jnp.dot(mean, wn_ref[...], preferred_element_type=jnp.float32)
        + jnp.dot(x_ref[...], ws_ref[...], preferred_element_type=jnp.float32)
        + b_ref[...], 0.0)
    a_ref[...] = jnp.dot(h, wq_ref[:D], preferred_element_type=jnp.float32)
    bmat = jnp.dot(h, wq_ref[D:], preferred_element_type=jnp.float32) + bq_ref[...]
    btab_ref[0] = bmat[:, :DW]
    btab_ref[1] = bmat[:, DW:]


def _final_body(a_ref, s_ref, q_ref, cs_ref, out_ref):
    a = a_ref[...]
    cs = cs_ref[1, :, 0:1]
    sb = jnp.concatenate([s_ref[0], s_ref[1]], axis=1)
    sb2 = jnp.concatenate([q_ref[0], q_ref[1]], axis=1)
    gs = cs * a * a + 2.0 * a * sb + sb2
    out_ref[...] = jnp.tanh(gs / jnp.maximum(cs, 1.0))


def kernel(X, edge_index, Wn, Ws, b_sage, Wq, bq):
    src = edge_index[0]
    dst = edge_index[1]

    pad = jnp.full((PC - E,), N, jnp.int32)
    srcp = jnp.concatenate([src, pad]).reshape(NS, CPT, CH)
    dstp = jnp.concatenate([dst, pad]).reshape(NS, CPT, CH)

    t1 = jnp.zeros((NC, NPAD, DW), jnp.float32)
    t1 = t1.at[0, :N].set(X[:, :DW]).at[1, :N].set(X[:, DW:])
    t1 = t1.reshape(NC * NPAD, DW)

    ones_rows = jnp.zeros((CH, CW), jnp.float32).at[:, 0].set(1.0)

    sweep1 = _make_sweep(DW, aux=True)
    o1d, o1c = sweep1(srcp, dstp, t1, ones_rows)

    grid = (NPAD // BN,)
    a_full, btab = pl.pallas_call(
        _dense_body,
        grid=grid,
        in_specs=[
            pl.BlockSpec((NC, BN, DW), lambda i: (0, i, 0)),
            pl.BlockSpec((NC, BN, CW), lambda i: (0, i, 0)),
            pl.BlockSpec((BN, D), lambda i: (i, 0)),
            pl.BlockSpec((D, D), lambda i: (0, 0)),
            pl.BlockSpec((D, D), lambda i: (0, 0)),
            pl.BlockSpec((1, D), lambda i: (0, 0)),
            pl.BlockSpec((2 * D, D), lambda i: (0, 0)),
            pl.BlockSpec((1, D), lambda i: (0, 0)),
        ],
        out_specs=[
            pl.BlockSpec((BN, D), lambda i: (i, 0)),
            pl.BlockSpec((NC, BN, DW), lambda i: (0, i, 0)),
        ],
        out_shape=[
            jax.ShapeDtypeStruct((NPAD, D), jnp.float32),
            jax.ShapeDtypeStruct((NC, NPAD, DW), jnp.float32),
        ],
    )(o1d, o1c, jnp.zeros((NPAD, D), jnp.float32).at[:N].set(X),
      Wn, Ws, b_sage.reshape(1, D), Wq, bq.reshape(1, D))

    sweep2 = _make_sweep(DW, aux=False, sq=True)
    o2b, o2q = sweep2(dstp, srcp, btab.reshape(NC * NPAD, DW))

    gg = pl.pallas_call(
        _final_body,
        grid=grid,
        in_specs=[
            pl.BlockSpec((BN, D), lambda i: (i, 0)),
            pl.BlockSpec((NC, BN, DW), lambda i: (0, i, 0)),
            pl.BlockSpec((NC, BN, DW), lambda i: (0, i, 0)),
            pl.BlockSpec((NC, BN, CW), lambda i: (0, i, 0)),
        ],
        out_specs=pl.BlockSpec((BN, D), lambda i: (i, 0)),
        out_shape=jax.ShapeDtypeStruct((NPAD, D), jnp.float32),
    )(a_full, o2b, o2q, o1c)

    return gg[:N]


# speedup vs baseline: 1.2868x; 1.0123x over previous
import functools

import jax
import jax.numpy as jnp
from jax import lax
from jax.experimental import pallas as pl
from jax.experimental.pallas import tpu as pltpu
from jax.experimental.pallas import tpu_sc as plsc

N = 10000
D = 128
E = 320000

NC = 2
NS = 16
CH = 128
DW = 64
CW = 16
NPAD = 10240
RPT = NPAD // NS
SEGC = 16


def _ceil_to(x, m):
    return (x + m - 1) // m * m


CPT = _ceil_to(_ceil_to(E // NS, CH) // CH, SEGC)
PC = CPT * CH * NS


def _make_sweep(dw, aux, sq=False):
    mesh = plsc.VectorSubcoreMesh(core_axis_name="c", subcore_axis_name="s")
    out_type = [jax.ShapeDtypeStruct((NC, NPAD, dw), jnp.float32)] * (
        2 if sq else 1)
    scratch = [
        pltpu.VMEM((SEGC, CH), jnp.int32),
        pltpu.VMEM((SEGC, CH), jnp.int32),
        pltpu.VMEM((CH, dw), jnp.float32),
        pltpu.VMEM((CH, dw), jnp.float32),
        pltpu.VMEM((16, dw), jnp.float32),
        pltpu.VMEM_SHARED((NPAD, dw), jnp.float32),
        pltpu.SemaphoreType.DMA,
        pltpu.SemaphoreType.DMA,
        pltpu.SemaphoreType.DMA,
        pltpu.SemaphoreType.DMA,
    ]
    if sq:
        scratch += [
            pltpu.VMEM((CH, dw), jnp.float32),
            pltpu.VMEM((CH, dw), jnp.float32),
            pltpu.VMEM_SHARED((NPAD, dw), jnp.float32),
            pltpu.SemaphoreType.DMA,
            pltpu.SemaphoreType.DMA,
        ]
    if aux:
        out_type += [jax.ShapeDtypeStruct((NC, NPAD, CW), jnp.float32)]
        scratch += [
            pltpu.VMEM((CH, CW), jnp.float32),
            pltpu.VMEM((16, CW), jnp.float32),
            pltpu.VMEM_SHARED((NPAD, CW), jnp.float32),
            pltpu.SemaphoreType.DMA,
        ]

    @functools.partial(
        pl.kernel,
        out_type=out_type,
        mesh=mesh,
        scratch_types=scratch,
        compiler_params=pltpu.CompilerParams(use_tc_tiling_on_sc=False),
    )
    def sweep(gidx_hbm, sidx_hbm, table_hbm, *rest):
        if aux:
            (ones_hbm, out_hbm, oc_hbm, gidx_v, sidx_v, rowbuf0,
             rowbuf1, ztile, acc, gsem0, gsem1, ssem0, ssem1, onesbuf, zc,
             acc_c, csem) = rest
        elif sq:
            (out_hbm, oq_hbm, gidx_v, sidx_v, rowbuf0, rowbuf1, ztile,
             acc, gsem0, gsem1, ssem0, ssem1, sqbuf0, sqbuf1, acc_q,
             qsem0, qsem1) = rest
            sqbufs = (sqbuf0, sqbuf1)
            qsems = (qsem0, qsem1)
        else:
            (out_hbm, gidx_v, sidx_v, rowbuf0, rowbuf1, ztile,
             acc, gsem0, gsem1, ssem0, ssem1) = rest
        rowbufs = (rowbuf0, rowbuf1)
        gsems = (gsem0, gsem1)
        ssems = (ssem0, ssem1)
        c = lax.axis_index("c")
        s = lax.axis_index("s")
        base = s * RPT
        tbl_c = table_hbm.at[pl.ds(c * NPAD, NPAD)]

        zv = jnp.zeros((16,), jnp.float32)
        for r in range(16):
            for o in range(0, dw, 16):
                ztile[r, pl.ds(o, 16)] = zv
        zpend = [
            pltpu.async_copy(ztile, acc.at[pl.ds(base + i * 16, 16)], gsem0)
            for i in range(RPT // 16)
        ]
        if sq:
            zpend += [
                pltpu.async_copy(ztile, acc_q.at[pl.ds(base + i * 16, 16)],
                                 gsem1)
                for i in range(RPT // 16)
            ]
        if aux:
            for r in range(16):
                zc[r, pl.ds(0, CW)] = zv
            zpend += [pltpu.async_copy(ones_hbm, onesbuf, csem)]
            zpend += [
                pltpu.async_copy(zc, acc_c.at[pl.ds(base + i * 16, 16)],
                                 gsem1)
                for i in range(RPT // 16)
            ]
        for zp in zpend:
            zp.wait()
        plsc.subcore_barrier()

        def seg_body(si, carry):
            ip0 = pltpu.async_copy(
                gidx_hbm.at[s, pl.ds(si * SEGC, SEGC)], gidx_v, gsem0)
            ip1 = pltpu.async_copy(
                sidx_hbm.at[s, pl.ds(si * SEGC, SEGC)], sidx_v, gsem1)
            ip0.wait()
            ip1.wait()

            pend_g = [None, None]
            pend_s = [None, None]
            pend_q = [None, None]
            pend_g[0] = pltpu.async_copy(
                tbl_c.at[gidx_v.at[0]], rowbufs[0], gsems[0])
            for j in range(SEGC):
                cur = j & 1
                if j + 1 < SEGC:
                    if pend_s[1 - cur] is not None:
                        pend_s[1 - cur].wait()
                    if sq and pend_q[1 - cur] is not None:
                        pend_q[1 - cur].wait()
                    pend_g[1 - cur] = pltpu.async_copy(
                        tbl_c.at[gidx_v.at[j + 1]],
                        rowbufs[1 - cur], gsems[1 - cur])
                pend_g[cur].wait()
                if sq:
                    rb, qb = rowbufs[cur], sqbufs[cur]

                    @plsc.parallel_loop(0, CH, unroll=4)
                    def _(r):
                        for o in range(0, dw, 16):
                            v = rb[r, pl.ds(o, 16)]
                            qb[r, pl.ds(o, 16)] = v * v
                    pend_q[cur] = pltpu.async_copy(
                        qb, acc_q.at[sidx_v.at[j]], qsems[cur], add=True)
                pend_s[cur] = pltpu.async_copy(
                    rowbufs[cur], acc.at[sidx_v.at[j]], ssems[cur],
                    add=True)
                if aux:
                    @pl.when(c == 0)
                    def _():
                        pltpu.async_copy(
                            onesbuf, acc_c.at[sidx_v.at[j]], csem, add=True)

                    @pl.when(c == 1)
                    def _():
                        pltpu.async_copy(
                            onesbuf, acc_c.at[gidx_v.at[j]], csem, add=True)
            pend_s[0].wait()
            pend_s[1].wait()
            if sq:
                if pend_q[0] is not None:
                    pend_q[0].wait()
                if pend_q[1] is not None:
                    pend_q[1].wait()
            if aux:
                for j in range(SEGC):
                    pltpu.make_async_copy(
                        onesbuf, acc_c.at[sidx_v.at[0]], csem).wait()
            return carry
        lax.fori_loop(0, CPT // SEGC, seg_body, 0)
        plsc.subcore_barrier()

        pltpu.sync_copy(acc.at[pl.ds(base, RPT)],
                        out_hbm.at[c, pl.ds(base, RPT)])
        if sq:
            pltpu.sync_copy(acc_q.at[pl.ds(base, RPT)],
                            oq_hbm.at[c, pl.ds(base, RPT)])
        if aux:
            pltpu.sync_copy(acc_c.at[pl.ds(base, RPT)],
                            oc_hbm.at[c, pl.ds(base, RPT)])

    return sweep


BN = 1024


def _dense_body(agg_ref, cnt_ref, x_ref, wn_ref, ws_ref, b_ref, wq_ref,
                bq_ref, a_ref, btab_ref):
    agg = jnp.concatenate([agg_ref[0], agg_ref[1]], axis=1)
    cnt = cnt_ref[0, :, 0:1]
    mean = agg / jnp.maximum(cnt, 1.0)
    h = jnp.maximum(
        jnp.dot(mean, wn_ref[...], preferred_element_type=jnp.float32)
        + jnp.dot(x_ref[...], ws_ref[...], preferred_element_type=jnp.float32)
        + b_ref[...], 0.0)
    a_ref[...] = jnp.dot(h, wq_ref[:D], preferred_element_type=jnp.float32)
    bmat = jnp.dot(h, wq_ref[D:], preferred_element_type=jnp.float32) + bq_ref[...]
    btab_ref[0] = bmat[:, :DW]
    btab_ref[1] = bmat[:, DW:]


def _final_body(a_ref, s_ref, q_ref, cs_ref, out_ref):
    a = a_ref[...]
    cs = cs_ref[1, :, 0:1]
    sb = jnp.concatenate([s_ref[0], s_ref[1]], axis=1)
    sb2 = jnp.concatenate([q_ref[0], q_ref[1]], axis=1)
    gs = cs * a * a + 2.0 * a * sb + sb2
    out_ref[...] = jnp.tanh(gs / jnp.maximum(cs, 1.0))


def kernel(X, edge_index, Wn, Ws, b_sage, Wq, bq):
    src = edge_index[0]
    dst = edge_index[1]

    pad = jnp.full((PC - E,), N, jnp.int32)
    srcp = jnp.concatenate([src, pad]).reshape(NS, CPT, CH)
    dstp = jnp.concatenate([dst, pad]).reshape(NS, CPT, CH)

    t1 = jnp.zeros((NC, NPAD, DW), jnp.float32)
    t1 = t1.at[0, :N].set(X[:, :DW]).at[1, :N].set(X[:, DW:])
    t1 = t1.reshape(NC * NPAD, DW)

    ones_rows = jnp.zeros((CH, CW), jnp.float32).at[:, 0].set(1.0)

    sweep1 = _make_sweep(DW, aux=True)
    o1d, o1c = sweep1(srcp, dstp, t1, ones_rows)

    grid = (NPAD // BN,)
    a_full, btab = pl.pallas_call(
        _dense_body,
        grid=grid,
        in_specs=[
            pl.BlockSpec((NC, BN, DW), lambda i: (0, i, 0)),
            pl.BlockSpec((NC, BN, CW), lambda i: (0, i, 0)),
            pl.BlockSpec((BN, D), lambda i: (i, 0)),
            pl.BlockSpec((D, D), lambda i: (0, 0)),
            pl.BlockSpec((D, D), lambda i: (0, 0)),
            pl.BlockSpec((1, D), lambda i: (0, 0)),
            pl.BlockSpec((2 * D, D), lambda i: (0, 0)),
            pl.BlockSpec((1, D), lambda i: (0, 0)),
        ],
        out_specs=[
            pl.BlockSpec((BN, D), lambda i: (i, 0)),
            pl.BlockSpec((NC, BN, DW), lambda i: (0, i, 0)),
        ],
        out_shape=[
            jax.ShapeDtypeStruct((NPAD, D), jnp.float32),
            jax.ShapeDtypeStruct((NC, NPAD, DW), jnp.float32),
        ],
    )(o1d, o1c, jnp.zeros((NPAD, D), jnp.float32).at[:N].set(X),
      Wn, Ws, b_sage.reshape(1, D), Wq, bq.reshape(1, D))

    sweep2 = _make_sweep(DW, aux=False, sq=True)
    o2b, o2q = sweep2(dstp, srcp, btab.reshape(NC * NPAD, DW))

    gg = pl.pallas_call(
        _final_body,
        grid=grid,
        in_specs=[
            pl.BlockSpec((BN, D), lambda i: (i, 0)),
            pl.BlockSpec((NC, BN, DW), lambda i: (0, i, 0)),
            pl.BlockSpec((NC, BN, DW), lambda i: (0, i, 0)),
            pl.BlockSpec((NC, BN, CW), lambda i: (0, i, 0)),
        ],
        out_specs=pl.BlockSpec((BN, D), lambda i: (i, 0)),
        out_shape=jax.ShapeDtypeStruct((NPAD, D), jnp.float32),
    )(a_full, o2b, o2q, o1c)

    return gg[:N]


# speedup vs baseline: 1.3153x; 1.0222x over previous
import functools

import jax
import jax.numpy as jnp
from jax import lax
from jax.experimental import pallas as pl
from jax.experimental.pallas import tpu as pltpu
from jax.experimental.pallas import tpu_sc as plsc

N = 10000
D = 128
E = 320000

NC = 2
NS = 16
CH = 128
DW = 64
CW = 16
NPAD = 10240
RPT = NPAD // NS
SEGC = 32


def _ceil_to(x, m):
    return (x + m - 1) // m * m


CPT = _ceil_to(_ceil_to(E // NS, CH) // CH, SEGC)
PC = CPT * CH * NS


def _make_sweep(dw, aux, sq=False):
    mesh = plsc.VectorSubcoreMesh(core_axis_name="c", subcore_axis_name="s")
    out_type = [jax.ShapeDtypeStruct((NC, NPAD, dw), jnp.float32)] * (
        2 if sq else 1)
    scratch = [
        pltpu.VMEM((SEGC, CH), jnp.int32),
        pltpu.VMEM((SEGC, CH), jnp.int32),
        pltpu.VMEM((CH, dw), jnp.float32),
        pltpu.VMEM((CH, dw), jnp.float32),
        pltpu.VMEM((16, dw), jnp.float32),
        pltpu.VMEM_SHARED((NPAD, dw), jnp.float32),
        pltpu.SemaphoreType.DMA,
        pltpu.SemaphoreType.DMA,
        pltpu.SemaphoreType.DMA,
        pltpu.SemaphoreType.DMA,
    ]
    if sq:
        scratch += [
            pltpu.VMEM((CH, dw), jnp.float32),
            pltpu.VMEM((CH, dw), jnp.float32),
            pltpu.VMEM_SHARED((NPAD, dw), jnp.float32),
            pltpu.SemaphoreType.DMA,
            pltpu.SemaphoreType.DMA,
        ]
    if aux:
        out_type += [jax.ShapeDtypeStruct((NC, NPAD, CW), jnp.float32)]
        scratch += [
            pltpu.VMEM((CH, CW), jnp.float32),
            pltpu.VMEM((16, CW), jnp.float32),
            pltpu.VMEM_SHARED((NPAD, CW), jnp.float32),
            pltpu.SemaphoreType.DMA,
        ]

    @functools.partial(
        pl.kernel,
        out_type=out_type,
        mesh=mesh,
        scratch_types=scratch,
        compiler_params=pltpu.CompilerParams(use_tc_tiling_on_sc=False),
    )
    def sweep(gidx_hbm, sidx_hbm, table_hbm, *rest):
        if aux:
            (ones_hbm, out_hbm, oc_hbm, gidx_v, sidx_v, rowbuf0,
             rowbuf1, ztile, acc, gsem0, gsem1, ssem0, ssem1, onesbuf, zc,
             acc_c, csem) = rest
        elif sq:
            (out_hbm, oq_hbm, gidx_v, sidx_v, rowbuf0, rowbuf1, ztile,
             acc, gsem0, gsem1, ssem0, ssem1, sqbuf0, sqbuf1, acc_q,
             qsem0, qsem1) = rest
            sqbufs = (sqbuf0, sqbuf1)
            qsems = (qsem0, qsem1)
        else:
            (out_hbm, gidx_v, sidx_v, rowbuf0, rowbuf1, ztile,
             acc, gsem0, gsem1, ssem0, ssem1) = rest
        rowbufs = (rowbuf0, rowbuf1)
        gsems = (gsem0, gsem1)
        ssems = (ssem0, ssem1)
        c = lax.axis_index("c")
        s = lax.axis_index("s")
        base = s * RPT
        tbl_c = table_hbm.at[pl.ds(c * NPAD, NPAD)]

        zv = jnp.zeros((16,), jnp.float32)
        for r in range(16):
            for o in range(0, dw, 16):
                ztile[r, pl.ds(o, 16)] = zv
        zpend = [
            pltpu.async_copy(ztile, acc.at[pl.ds(base + i * 16, 16)], gsem0)
            for i in range(RPT // 16)
        ]
        if sq:
            zpend += [
                pltpu.async_copy(ztile, acc_q.at[pl.ds(base + i * 16, 16)],
                                 gsem1)
                for i in range(RPT // 16)
            ]
        if aux:
            for r in range(16):
                zc[r, pl.ds(0, CW)] = zv
            zpend += [pltpu.async_copy(ones_hbm, onesbuf, csem)]
            zpend += [
                pltpu.async_copy(zc, acc_c.at[pl.ds(base + i * 16, 16)],
                                 gsem1)
                for i in range(RPT // 16)
            ]
        for zp in zpend:
            zp.wait()
        plsc.subcore_barrier()

        def seg_body(si, carry):
            ip0 = pltpu.async_copy(
                gidx_hbm.at[s, pl.ds(si * SEGC, SEGC)], gidx_v, gsem0)
            ip1 = pltpu.async_copy(
                sidx_hbm.at[s, pl.ds(si * SEGC, SEGC)], sidx_v, gsem1)
            ip0.wait()
            ip1.wait()

            pend_g = [None, None]
            pend_s = [None, None]
            pend_q = [None, None]
            pend_g[0] = pltpu.async_copy(
                tbl_c.at[gidx_v.at[0]], rowbufs[0], gsems[0])
            for j in range(SEGC):
                cur = j & 1
                if j + 1 < SEGC:
                    if pend_s[1 - cur] is not None:
                        pend_s[1 - cur].wait()
                    if sq and pend_q[1 - cur] is not None:
                        pend_q[1 - cur].wait()
                    pend_g[1 - cur] = pltpu.async_copy(
                        tbl_c.at[gidx_v.at[j + 1]],
                        rowbufs[1 - cur], gsems[1 - cur])
                pend_g[cur].wait()
                if sq:
                    rb, qb = rowbufs[cur], sqbufs[cur]

                    @plsc.parallel_loop(0, CH, unroll=4)
                    def _(r):
                        for o in range(0, dw, 16):
                            v = rb[r, pl.ds(o, 16)]
                            qb[r, pl.ds(o, 16)] = v * v
                    pend_q[cur] = pltpu.async_copy(
                        qb, acc_q.at[sidx_v.at[j]], qsems[cur], add=True)
                pend_s[cur] = pltpu.async_copy(
                    rowbufs[cur], acc.at[sidx_v.at[j]], ssems[cur],
                    add=True)
                if aux:
                    @pl.when(c == 0)
                    def _():
                        pltpu.async_copy(
                            onesbuf, acc_c.at[sidx_v.at[j]], csem, add=True)

                    @pl.when(c == 1)
                    def _():
                        pltpu.async_copy(
                            onesbuf, acc_c.at[gidx_v.at[j]], csem, add=True)
            pend_s[0].wait()
            pend_s[1].wait()
            if sq:
                if pend_q[0] is not None:
                    pend_q[0].wait()
                if pend_q[1] is not None:
                    pend_q[1].wait()
            if aux:
                for j in range(SEGC):
                    pltpu.make_async_copy(
                        onesbuf, acc_c.at[sidx_v.at[0]], csem).wait()
            return carry
        lax.fori_loop(0, CPT // SEGC, seg_body, 0)
        plsc.subcore_barrier()

        pltpu.sync_copy(acc.at[pl.ds(base, RPT)],
                        out_hbm.at[c, pl.ds(base, RPT)])
        if sq:
            pltpu.sync_copy(acc_q.at[pl.ds(base, RPT)],
                            oq_hbm.at[c, pl.ds(base, RPT)])
        if aux:
            pltpu.sync_copy(acc_c.at[pl.ds(base, RPT)],
                            oc_hbm.at[c, pl.ds(base, RPT)])

    return sweep


BN = 1024


def _dense_body(agg_ref, cnt_ref, x_ref, wn_ref, ws_ref, b_ref, wq_ref,
                bq_ref, a_ref, btab_ref):
    agg = jnp.concatenate([agg_ref[0], agg_ref[1]], axis=1)
    cnt = cnt_ref[0, :, 0:1]
    mean = agg / jnp.maximum(cnt, 1.0)
    h = jnp.maximum(
        jnp.dot(mean, wn_ref[...], preferred_element_type=jnp.float32)
        + jnp.dot(x_ref[...], ws_ref[...], preferred_element_type=jnp.float32)
        + b_ref[...], 0.0)
    a_ref[...] = jnp.dot(h, wq_ref[:D], preferred_element_type=jnp.float32)
    bmat = jnp.dot(h, wq_ref[D:], preferred_element_type=jnp.float32) + bq_ref[...]
    btab_ref[0] = bmat[:, :DW]
    btab_ref[1] = bmat[:, DW:]


def _final_body(a_ref, s_ref, q_ref, cs_ref, out_ref):
    a = a_ref[...]
    cs = cs_ref[1, :, 0:1]
    sb = jnp.concatenate([s_ref[0], s_ref[1]], axis=1)
    sb2 = jnp.concatenate([q_ref[0], q_ref[1]], axis=1)
    gs = cs * a * a + 2.0 * a * sb + sb2
    out_ref[...] = jnp.tanh(gs / jnp.maximum(cs, 1.0))


def kernel(X, edge_index, Wn, Ws, b_sage, Wq, bq):
    src = edge_index[0]
    dst = edge_index[1]

    pad = jnp.full((PC - E,), N, jnp.int32)
    srcp = jnp.concatenate([src, pad]).reshape(NS, CPT, CH)
    dstp = jnp.concatenate([dst, pad]).reshape(NS, CPT, CH)

    t1 = jnp.zeros((NC, NPAD, DW), jnp.float32)
    t1 = t1.at[0, :N].set(X[:, :DW]).at[1, :N].set(X[:, DW:])
    t1 = t1.reshape(NC * NPAD, DW)

    ones_rows = jnp.zeros((CH, CW), jnp.float32).at[:, 0].set(1.0)

    sweep1 = _make_sweep(DW, aux=True)
    o1d, o1c = sweep1(srcp, dstp, t1, ones_rows)

    grid = (NPAD // BN,)
    a_full, btab = pl.pallas_call(
        _dense_body,
        grid=grid,
        in_specs=[
            pl.BlockSpec((NC, BN, DW), lambda i: (0, i, 0)),
            pl.BlockSpec((NC, BN, CW), lambda i: (0, i, 0)),
            pl.BlockSpec((BN, D), lambda i: (i, 0)),
            pl.BlockSpec((D, D), lambda i: (0, 0)),
            pl.BlockSpec((D, D), lambda i: (0, 0)),
            pl.BlockSpec((1, D), lambda i: (0, 0)),
            pl.BlockSpec((2 * D, D), lambda i: (0, 0)),
            pl.BlockSpec((1, D), lambda i: (0, 0)),
        ],
        out_specs=[
            pl.BlockSpec((BN, D), lambda i: (i, 0)),
            pl.BlockSpec((NC, BN, DW), lambda i: (0, i, 0)),
        ],
        out_shape=[
            jax.ShapeDtypeStruct((NPAD, D), jnp.float32),
            jax.ShapeDtypeStruct((NC, NPAD, DW), jnp.float32),
        ],
    )(o1d, o1c, jnp.zeros((NPAD, D), jnp.float32).at[:N].set(X),
      Wn, Ws, b_sage.reshape(1, D), Wq, bq.reshape(1, D))

    sweep2 = _make_sweep(DW, aux=False, sq=True)
    o2b, o2q = sweep2(dstp, srcp, btab.reshape(NC * NPAD, DW))

    gg = pl.pallas_call(
        _final_body,
        grid=grid,
        in_specs=[
            pl.BlockSpec((BN, D), lambda i: (i, 0)),
            pl.BlockSpec((NC, BN, DW), lambda i: (0, i, 0)),
            pl.BlockSpec((NC, BN, DW), lambda i: (0, i, 0)),
            pl.BlockSpec((NC, BN, CW), lambda i: (0, i, 0)),
        ],
        out_specs=pl.BlockSpec((BN, D), lambda i: (i, 0)),
        out_shape=jax.ShapeDtypeStruct((NPAD, D), jnp.float32),
    )(a_full, o2b, o2q, o1c)

    return gg[:N]
